# baseline (speedup 1.0000x reference)
import functools

import jax
import jax.numpy as jnp
from jax import lax
from jax.experimental import pallas as pl
from jax.experimental.pallas import tpu as pltpu
from jax.experimental.pallas import tpu_sc as plsc

B = 16384
C = 100
S_SCALE = 30.0
NC = 2
NS = 16
L = 16
NW = NC * NS
RPW = B // NW
WORDS = RPW * C
NGROUP = RPW // L

_LN2 = 0.6931471805599453
_SQRT2 = 1.4142135623730951

_DNUMS = lax.GatherDimensionNumbers(
    offset_dims=(), collapsed_slice_dims=(0,), start_index_map=(0,))


def _perm(v, idx):
    return lax.gather(v, idx[:, None], _DNUMS, slice_sizes=(1,),
                      mode=lax.GatherScatterMode.PROMISE_IN_BOUNDS)


def _ldam_body(x_hbm, t_hbm, m_hbm, out_hbm, xv, tv, mv, accv):
    wid = lax.axis_index("s") * NC + lax.axis_index("c")
    base = wid * RPW
    pltpu.sync_copy(x_hbm.at[pl.ds(base * C, WORDS)], xv)
    pltpu.sync_copy(t_hbm.at[pl.ds(base, RPW)], tv)
    pltpu.sync_copy(m_hbm, mv.at[pl.ds(0, C)])

    neg = jnp.float32(-3.0e38)
    zero = jnp.zeros((L,), jnp.float32)

    def allmax(v, perms):
        for p in perms:
            v = jnp.maximum(v, _perm(v, p))
        return v

    def allsum(v, perms):
        for p in perms:
            v = v + _perm(v, p)
        return v

    def pick_m(mreg, t_s):
        q = t_s >> 4
        cand = mreg[0]
        for k in range(1, 7):
            cand = jnp.where(q == k, mreg[k], cand)
        return _perm(cand, jnp.full((L,), t_s & 15, jnp.int32))

    def pick_a(v, t_s):
        q = t_s >> 4
        cand = v[0]
        for k in range(1, 7):
            cand = jnp.where(q == k, v[k], cand)
        return _perm(cand, jnp.full((L,), t_s & 15, jnp.int32))

    def pick_b(v, v12, t_s):
        off = t_s + 100
        q = off >> 4
        cand = v[6]
        for k in range(7, 12):
            cand = jnp.where(q == k, v[k], cand)
        high = off >= 192
        cand = jnp.where(high, v12, cand)
        lane_sc = jnp.where(high, off - 184, off & 15)
        return _perm(cand, jnp.full((L,), lane_sc, jnp.int32))

    def group_body(g, acc):
        lane = lax.iota(jnp.int32, L)
        perms = [lane ^ sh for sh in (8, 4, 2, 1)]
        mask_a6 = lane < 4
        mask_b12 = lane >= 8
        mreg = [mv[pl.ds(16 * k, 16)] for k in range(6)] + [mv[pl.ds(96, 16)]]
        r0 = g * L
        tt = tv[pl.ds(r0, L)]
        kk = zero
        ss = zero
        xt = zero
        mt = zero
        for j in range(8):
            b0 = (r0 + 2 * j) * C
            v = [xv[pl.ds(b0 + 16 * k, 16)] for k in range(12)]
            v12 = xv[pl.ds(b0 + 184, 16)]
            ma = v[0]
            for k in range(1, 6):
                ma = jnp.maximum(ma, v[k])
            ma = jnp.maximum(ma, jnp.where(mask_a6, v[6], neg))
            mb = v[7]
            for k in range(8, 12):
                mb = jnp.maximum(mb, v[k])
            mb = jnp.maximum(mb, jnp.where(mask_a6, neg, v[6]))
            mb = jnp.maximum(mb, jnp.where(mask_b12, v12, neg))
            ka = S_SCALE * allmax(ma, perms)
            kb = S_SCALE * allmax(mb, perms)
            sa = jnp.exp(S_SCALE * v[0] - ka)
            for k in range(1, 6):
                sa = sa + jnp.exp(S_SCALE * v[k] - ka)
            e6 = jnp.exp(S_SCALE * v[6] - jnp.where(mask_a6, ka, kb))
            sa = sa + jnp.where(mask_a6, e6, 0.0)
            sb = jnp.where(mask_a6, 0.0, e6)
            for k in range(7, 12):
                sb = sb + jnp.exp(S_SCALE * v[k] - kb)
            e12 = jnp.exp(S_SCALE * v12 - kb)
            sb = sb + jnp.where(mask_b12, e12, 0.0)
            suma = allsum(sa, perms)
            sumb = allsum(sb, perms)
            ta = tt[2 * j]
            tb = tt[2 * j + 1]
            xa = pick_a(v, ta)
            xb = pick_b(v, v12, tb)
            is_a = lane == 2 * j
            is_b = lane == 2 * j + 1
            kk = jnp.where(is_a, ka, jnp.where(is_b, kb, kk))
            ss = jnp.where(is_a, suma, jnp.where(is_b, sumb, ss))
            xt = jnp.where(is_a, xa, jnp.where(is_b, xb, xt))
            mt = jnp.where(is_a, pick_m(mreg, ta),
                           jnp.where(is_b, pick_m(mreg, tb), mt))

        zt = S_SCALE * xt
        ztm = zt - S_SCALE * mt
        smod = ss - jnp.exp(zt - kk) + jnp.exp(ztm - kk)
        bits = lax.bitcast_convert_type(smod, jnp.int32)
        ex = lax.shift_right_arithmetic(bits, 23) - 127
        mf = lax.bitcast_convert_type(
            lax.bitwise_or(lax.bitwise_and(bits, 0x7FFFFF), 0x3F800000),
            jnp.float32)
        big = mf > _SQRT2
        mf = jnp.where(big, mf * 0.5, mf)
        ex = jnp.where(big, ex + 1, ex)
        u = (mf - 1.0) / (mf + 1.0)
        u2 = u * u
        ln = ex.astype(jnp.float32) * _LN2 + 2.0 * u * (
            1.0 + u2 * (0.3333333333 + u2 * 0.2))
        nll = kk + ln - ztm
        return acc + nll * (1.0 / B)

    acc = lax.fori_loop(0, NGROUP, group_body, zero)
    accv[...] = acc
    pltpu.sync_copy(accv, out_hbm.at[wid])


_ldam_sc = functools.partial(
    pl.kernel,
    out_type=jax.ShapeDtypeStruct((NW, L), jnp.float32),
    mesh=plsc.VectorSubcoreMesh(core_axis_name="c", subcore_axis_name="s"),
    scratch_types=[
        pltpu.VMEM((WORDS,), jnp.float32),
        pltpu.VMEM((RPW,), jnp.int32),
        pltpu.VMEM((112,), jnp.float32),
        pltpu.VMEM((L,), jnp.float32),
    ],
)(_ldam_body)


def _sum_body(p_ref, o_ref):
    o_ref[0, 0] = jnp.sum(p_ref[...])


_sum_tc = pl.pallas_call(
    _sum_body,
    out_shape=jax.ShapeDtypeStruct((1, 1), jnp.float32),
    out_specs=pl.BlockSpec(memory_space=pltpu.SMEM),
)


def kernel(x, target, m_list):
    parts = _ldam_sc(x.reshape(B * C), target, m_list)
    return _sum_tc(parts)[0, 0]


# speedup vs baseline: 1.3803x; 1.3803x over previous
import functools

import jax
import jax.numpy as jnp
from jax import lax
from jax.experimental import pallas as pl
from jax.experimental.pallas import tpu as pltpu
from jax.experimental.pallas import tpu_sc as plsc

B = 16384
C = 100
S_SCALE = 30.0
NC = 2
NS = 16
L = 16
NW = NC * NS
RPW = B // NW
NGROUP = RPW // L

_LN2 = 0.6931471805599453
_SQRT2 = 1.4142135623730951

_DNUMS = lax.GatherDimensionNumbers(
    offset_dims=(), collapsed_slice_dims=(0,), start_index_map=(0,))


def _perm(v, idx):
    return lax.gather(v, idx[:, None], _DNUMS, slice_sizes=(1,),
                      mode=lax.GatherScatterMode.PROMISE_IN_BOUNDS)


def _ldam_body(x_hbm, t_hbm, m_hbm, out_hbm, xv, tv, mv, accv):
    wid = lax.axis_index("s") * NC + lax.axis_index("c")
    base = wid * RPW
    pltpu.sync_copy(x_hbm.at[pl.ds(base, RPW)], xv)
    pltpu.sync_copy(t_hbm.at[pl.ds(base, RPW)], tv)
    pltpu.sync_copy(m_hbm, mv.at[pl.ds(0, C)])

    zero = jnp.zeros((L,), jnp.float32)

    def allmax(v, perms):
        for p in perms:
            v = jnp.maximum(v, _perm(v, p))
        return v

    def allsum(v, perms):
        for p in perms:
            v = v + _perm(v, p)
        return v

    def pick_m(mreg, t_s):
        q = t_s >> 4
        cand = mreg[0]
        for k in range(1, 7):
            cand = jnp.where(q == k, mreg[k], cand)
        return _perm(cand, jnp.full((L,), t_s & 15, jnp.int32))

    def pick_x(v, t_s):
        q = t_s >> 4
        cand = v[0]
        for k in range(1, 6):
            cand = jnp.where(q == k, v[k], cand)
        high = t_s >= 96
        cand = jnp.where(high, v[6], cand)
        lane_sc = jnp.where(high, t_s - 84, t_s & 15)
        return _perm(cand, jnp.full((L,), lane_sc, jnp.int32))

    def group_body(g, acc):
        lane = lax.iota(jnp.int32, L)
        perms = [lane ^ sh for sh in (8, 4, 2, 1)]
        mask_tail = lane >= 12
        mreg = [mv[pl.ds(16 * k, 16)] for k in range(6)] + [mv[pl.ds(96, 16)]]
        r0 = g * L
        tt = tv[pl.ds(r0, L)]
        kk = zero
        ss = zero
        xt = zero
        mt = zero
        for j in range(L):
            r = r0 + j
            v = [xv[r, pl.ds(16 * k, 16)] for k in range(6)]
            v.append(xv[r, pl.ds(84, 16)])
            ma = jnp.maximum(v[0], v[1])
            for k in range(2, 7):
                ma = jnp.maximum(ma, v[k])
            ka = S_SCALE * allmax(ma, perms)
            sa = jnp.exp(S_SCALE * v[0] - ka)
            for k in range(1, 6):
                sa = sa + jnp.exp(S_SCALE * v[k] - ka)
            sa = sa + jnp.where(mask_tail, jnp.exp(S_SCALE * v[6] - ka), 0.0)
            suma = allsum(sa, perms)
            t_s = tt[j]
            is_r = lane == j
            kk = jnp.where(is_r, ka, kk)
            ss = jnp.where(is_r, suma, ss)
            xt = jnp.where(is_r, pick_x(v, t_s), xt)
            mt = jnp.where(is_r, pick_m(mreg, t_s), mt)

        zt = S_SCALE * xt
        ztm = zt - S_SCALE * mt
        smod = ss - jnp.exp(zt - kk) + jnp.exp(ztm - kk)
        bits = lax.bitcast_convert_type(smod, jnp.int32)
        ex = lax.shift_right_arithmetic(bits, 23) - 127
        mf = lax.bitcast_convert_type(
            lax.bitwise_or(lax.bitwise_and(bits, 0x7FFFFF), 0x3F800000),
            jnp.float32)
        big = mf > _SQRT2
        mf = jnp.where(big, mf * 0.5, mf)
        ex = jnp.where(big, ex + 1, ex)
        u = (mf - 1.0) / (mf + 1.0)
        u2 = u * u
        ln = ex.astype(jnp.float32) * _LN2 + 2.0 * u * (
            1.0 + u2 * (0.3333333333 + u2 * 0.2))
        nll = kk + ln - ztm
        return acc + nll * (1.0 / B)

    acc = lax.fori_loop(0, NGROUP, group_body, zero)
    accv[...] = acc
    pltpu.sync_copy(accv, out_hbm.at[wid])


_ldam_sc = functools.partial(
    pl.kernel,
    out_type=jax.ShapeDtypeStruct((NW, L), jnp.float32),
    mesh=plsc.VectorSubcoreMesh(core_axis_name="c", subcore_axis_name="s"),
    scratch_types=[
        pltpu.VMEM((RPW, C), jnp.float32),
        pltpu.VMEM((RPW,), jnp.int32),
        pltpu.VMEM((112,), jnp.float32),
        pltpu.VMEM((L,), jnp.float32),
    ],
)(_ldam_body)


def _sum_body(p_ref, o_ref):
    o_ref[0, 0] = jnp.sum(p_ref[...])


_sum_tc = pl.pallas_call(
    _sum_body,
    out_shape=jax.ShapeDtypeStruct((1, 1), jnp.float32),
    out_specs=pl.BlockSpec(memory_space=pltpu.SMEM),
)


def kernel(x, target, m_list):
    parts = _ldam_sc(x, target, m_list)
    return _sum_tc(parts)[0, 0]


# speedup vs baseline: 1.3834x; 1.0022x over previous
import functools

import jax
import jax.numpy as jnp
from jax import lax
from jax.experimental import pallas as pl
from jax.experimental.pallas import tpu as pltpu
from jax.experimental.pallas import tpu_sc as plsc

B = 16384
C = 100
S_SCALE = 30.0
NC = 2
NS = 16
L = 16
NW = NC * NS
RPW = B // NW
NGROUP = RPW // L

_LN2 = 0.6931471805599453
_SQRT2 = 1.4142135623730951

_DNUMS = lax.GatherDimensionNumbers(
    offset_dims=(), collapsed_slice_dims=(0,), start_index_map=(0,))


def _perm(v, idx):
    return lax.gather(v, idx[:, None], _DNUMS, slice_sizes=(1,),
                      mode=lax.GatherScatterMode.PROMISE_IN_BOUNDS)


def _ldam_body(x_hbm, t_hbm, m_hbm, out_hbm, xv, tv, mv, accv):
    wid = lax.axis_index("s") * NC + lax.axis_index("c")
    base = wid * RPW
    pltpu.sync_copy(x_hbm.at[pl.ds(base, RPW)], xv)
    pltpu.sync_copy(t_hbm.at[pl.ds(base, RPW)], tv)
    pltpu.sync_copy(m_hbm, mv.at[pl.ds(0, C)])

    zero = jnp.zeros((L,), jnp.float32)

    def allmax(v, perms):
        for p in perms:
            v = jnp.maximum(v, _perm(v, p))
        return v

    def allsum(v, perms):
        for p in perms:
            v = v + _perm(v, p)
        return v

    def pick_m(mreg, t_s):
        q = t_s >> 4
        cand = mreg[0]
        for k in range(1, 7):
            cand = jnp.where(q == k, mreg[k], cand)
        return _perm(cand, jnp.full((L,), t_s & 15, jnp.int32))

    def pick_x(v, t_s):
        q = t_s >> 4
        cand = v[0]
        for k in range(1, 6):
            cand = jnp.where(q == k, v[k], cand)
        high = t_s >= 96
        cand = jnp.where(high, v[6], cand)
        lane_sc = jnp.where(high, t_s - 84, t_s & 15)
        return _perm(cand, jnp.full((L,), lane_sc, jnp.int32))

    def group_body(g, acc):
        lane = lax.iota(jnp.int32, L)
        perms = [lane ^ sh for sh in (8, 4, 2, 1)]
        mask_tail = lane >= 12
        mreg = [mv[pl.ds(16 * k, 16)] for k in range(6)] + [mv[pl.ds(96, 16)]]
        r0 = g * L
        tt = tv[pl.ds(r0, L)]
        kk = zero
        ss = zero
        xt = zero
        mt = zero
        for j in range(L):
            r = r0 + j
            v = [xv[r, pl.ds(16 * k, 16)] for k in range(6)]
            v.append(xv[r, pl.ds(84, 16)])
            ma = jnp.maximum(v[0], v[1])
            for k in range(2, 7):
                ma = jnp.maximum(ma, v[k])
            ka = S_SCALE * allmax(ma, perms)
            sa = jnp.exp(S_SCALE * v[0] - ka)
            for k in range(1, 6):
                sa = sa + jnp.exp(S_SCALE * v[k] - ka)
            sa = sa + jnp.where(mask_tail, jnp.exp(S_SCALE * v[6] - ka), 0.0)
            suma = allsum(sa, perms)
            t_s = tt[j]
            is_r = lane == j
            kk = jnp.where(is_r, ka, kk)
            ss = jnp.where(is_r, suma, ss)
            xt = jnp.where(is_r, pick_x(v, t_s), xt)
            mt = jnp.where(is_r, pick_m(mreg, t_s), mt)

        zt = S_SCALE * xt
        ztm = zt - S_SCALE * mt
        smod = ss - jnp.exp(zt - kk) + jnp.exp(ztm - kk)
        bits = lax.bitcast_convert_type(smod, jnp.int32)
        ex = lax.shift_right_arithmetic(bits, 23) - 127
        mf = lax.bitcast_convert_type(
            lax.bitwise_or(lax.bitwise_and(bits, 0x7FFFFF), 0x3F800000),
            jnp.float32)
        big = mf > _SQRT2
        mf = jnp.where(big, mf * 0.5, mf)
        ex = jnp.where(big, ex + 1, ex)
        u = (mf - 1.0) / (mf + 1.0)
        u2 = u * u
        ln = ex.astype(jnp.float32) * _LN2 + 2.0 * u * (
            1.0 + u2 * (0.3333333333 + u2 * 0.2))
        nll = kk + ln - ztm
        return acc + nll * (1.0 / B)

    acc = lax.fori_loop(0, NGROUP, group_body, zero)
    accv[...] = acc
    pltpu.sync_copy(accv, out_hbm.at[wid])


_ldam_sc = functools.partial(
    pl.kernel,
    out_type=jax.ShapeDtypeStruct((NW, L), jnp.float32),
    mesh=plsc.VectorSubcoreMesh(core_axis_name="c", subcore_axis_name="s"),
    compiler_params=pltpu.CompilerParams(use_tc_tiling_on_sc=True),
    scratch_types=[
        pltpu.VMEM((RPW, C), jnp.float32),
        pltpu.VMEM((RPW,), jnp.int32),
        pltpu.VMEM((112,), jnp.float32),
        pltpu.VMEM((L,), jnp.float32),
    ],
)(_ldam_body)


def _sum_body(p_ref, o_ref):
    o_ref[0, 0] = jnp.sum(p_ref[...])


_sum_tc = pl.pallas_call(
    _sum_body,
    out_shape=jax.ShapeDtypeStruct((1, 1), jnp.float32),
    out_specs=pl.BlockSpec(memory_space=pltpu.SMEM),
)


def kernel(x, target, m_list):
    parts = _ldam_sc(x, target, m_list)
    return _sum_tc(parts)[0, 0]


# speedup vs baseline: 1.3868x; 1.0024x over previous
import functools

import jax
import jax.numpy as jnp
from jax import lax
from jax.experimental import pallas as pl
from jax.experimental.pallas import tpu as pltpu
from jax.experimental.pallas import tpu_sc as plsc

B = 16384
C = 100
S_SCALE = 30.0
NC = 2
NS = 16
L = 16
NW = NC * NS
RPW = B // NW
NGROUP = RPW // L

_LN2 = 0.6931471805599453
_SQRT2 = 1.4142135623730951


def _ldam_body(xt_hbm, t_hbm, m_hbm, out_hbm, xv, tv, mv, accv):
    wid = lax.axis_index("s") * NC + lax.axis_index("c")
    base = wid * RPW
    pltpu.sync_copy(xt_hbm.at[:, pl.ds(base, RPW)], xv)
    pltpu.sync_copy(t_hbm.at[pl.ds(base, RPW)], tv)
    pltpu.sync_copy(m_hbm, mv.at[pl.ds(0, C)])

    neg = jnp.float32(-3.0e38)
    zero = jnp.zeros((L,), jnp.float32)

    def group_body(g, acc):
        mreg = [mv[pl.ds(16 * k, 16)] for k in range(7)]
        r0 = g * L
        tt = tv[pl.ds(r0, L)]
        mx = xv[0, pl.ds(r0, L)]
        for c in range(1, C):
            mx = jnp.maximum(mx, xv[c, pl.ds(r0, L)])
        kk = S_SCALE * mx
        ss = zero
        ztm = zero
        for c in range(C):
            v = xv[c, pl.ds(r0, L)]
            mc = mreg[c >> 4][c & 15]
            sel = tt == c
            w = S_SCALE * jnp.where(sel, v - mc, v)
            ss = ss + jnp.exp(w - kk)
            ztm = jnp.where(sel, w, ztm)
        bits = lax.bitcast_convert_type(ss, jnp.int32)
        ex = lax.shift_right_arithmetic(bits, 23) - 127
        mf = lax.bitcast_convert_type(
            lax.bitwise_or(lax.bitwise_and(bits, 0x7FFFFF), 0x3F800000),
            jnp.float32)
        big = mf > _SQRT2
        mf = jnp.where(big, mf * 0.5, mf)
        ex = jnp.where(big, ex + 1, ex)
        u = (mf - 1.0) / (mf + 1.0)
        u2 = u * u
        ln = ex.astype(jnp.float32) * _LN2 + 2.0 * u * (
            1.0 + u2 * (0.3333333333 + u2 * 0.2))
        nll = kk + ln - ztm
        return acc + nll * (1.0 / B)

    acc = lax.fori_loop(0, NGROUP, group_body, zero)
    accv[...] = acc
    pltpu.sync_copy(accv, out_hbm.at[wid])


_ldam_sc = functools.partial(
    pl.kernel,
    out_type=jax.ShapeDtypeStruct((NW, L), jnp.float32),
    mesh=plsc.VectorSubcoreMesh(core_axis_name="c", subcore_axis_name="s"),
    compiler_params=pltpu.CompilerParams(use_tc_tiling_on_sc=True),
    scratch_types=[
        pltpu.VMEM((C, RPW), jnp.float32),
        pltpu.VMEM((RPW,), jnp.int32),
        pltpu.VMEM((112,), jnp.float32),
        pltpu.VMEM((L,), jnp.float32),
    ],
)(_ldam_body)


def _sum_body(p_ref, o_ref):
    o_ref[0, 0] = jnp.sum(p_ref[...])


_sum_tc = pl.pallas_call(
    _sum_body,
    out_shape=jax.ShapeDtypeStruct((1, 1), jnp.float32),
    out_specs=pl.BlockSpec(memory_space=pltpu.SMEM),
)


def kernel(x, target, m_list):
    parts = _ldam_sc(x.T, target, m_list)
    return _sum_tc(parts)[0, 0]
